# Initial kernel scaffold; baseline (speedup 1.0000x reference)
import functools

import jax
import jax.numpy as jnp
from jax import lax
from jax.experimental import pallas as pl
from jax.experimental.pallas import tpu as pltpu
from jax.experimental.pallas import tpu_sc as plsc

N = 10000
E = 320000
D = 128

NC = 2
NS = 16
NW = NC * NS
EPW = E // NW
CH = 128
NFULL = EPW // CH
REM = EPW - NFULL * CH
RPT = N // NS
V = 16

_mesh = plsc.VectorSubcoreMesh(core_axis_name="c", subcore_axis_name="s")


def _deg_body(src_hbm, dst_hbm, deg_out, idx_s, idx_d, cnt_s, cnt_d):
    c = lax.axis_index("c")
    s = lax.axis_index("s")
    w = s * NC + c
    base = w * EPW
    pltpu.sync_copy(src_hbm.at[pl.ds(base, EPW)], idx_s)
    pltpu.sync_copy(dst_hbm.at[pl.ds(base, EPW)], idx_d)

    zeros = jnp.zeros((V,), jnp.float32)

    def _zero(i, _):
        cnt_s[pl.ds(i * V, V)] = zeros
        cnt_d[pl.ds(i * V, V)] = zeros
        return 0

    lax.fori_loop(0, N // V, _zero, 0)

    ones = jnp.ones((V,), jnp.float32)

    def _count(i, _):
        plsc.addupdate_scatter(cnt_s, [idx_s[pl.ds(i * V, V)]], ones)
        plsc.addupdate_scatter(cnt_d, [idx_d[pl.ds(i * V, V)]], ones)
        return 0

    lax.fori_loop(0, EPW // V, _count, 0)
    pltpu.sync_copy(cnt_s, deg_out.at[0, w])
    pltpu.sync_copy(cnt_d, deg_out.at[1, w])


_deg_call = pl.kernel(
    _deg_body,
    out_type=jax.ShapeDtypeStruct((2, NW, N), jnp.float32),
    mesh=_mesh,
    scratch_types=[
        pltpu.VMEM((EPW,), jnp.int32),
        pltpu.VMEM((EPW,), jnp.int32),
        pltpu.VMEM((N,), jnp.float32),
        pltpu.VMEM((N,), jnp.float32),
    ],
)


def _agg_body(q_hbm, src_hbm, dst_hbm, agg_out,
              sidx, didx, sidx_r, didx_r, rows, zbuf, acc, sem):
    c = lax.axis_index("c")
    s = lax.axis_index("s")
    w = s * NC + c

    zeros = jnp.zeros((V,), jnp.float32)

    def _z0(i, _):
        def _z1(j, _):
            zbuf[i, pl.ds(j * V, V)] = zeros
            return 0
        lax.fori_loop(0, D // V, _z1, 0)
        return 0

    lax.fori_loop(0, 125, _z0, 0)
    for k in range(RPT // 125):
        pltpu.sync_copy(zbuf.at[pl.ds(0, 125)],
                        acc.at[pl.ds(s * RPT + k * 125, 125)])
    plsc.subcore_barrier()

    base_w = w * EPW

    def _chunk(j, _):
        base = base_w + j * CH
        pltpu.sync_copy(src_hbm.at[pl.ds(base, CH)], sidx)
        pltpu.sync_copy(dst_hbm.at[pl.ds(base, CH)], didx)
        pltpu.async_copy(q_hbm.at[sidx], rows, sem).wait()
        pltpu.sync_copy(rows, acc.at[didx], add=True)
        return 0

    lax.fori_loop(0, NFULL, _chunk, 0)

    rbase = base_w + NFULL * CH
    pltpu.sync_copy(src_hbm.at[pl.ds(rbase, REM)], sidx_r)
    pltpu.sync_copy(dst_hbm.at[pl.ds(rbase, REM)], didx_r)
    pltpu.async_copy(q_hbm.at[sidx_r], rows.at[pl.ds(0, REM)], sem).wait()
    pltpu.sync_copy(rows.at[pl.ds(0, REM)], acc.at[didx_r], add=True)

    plsc.subcore_barrier()
    pltpu.sync_copy(acc.at[pl.ds(s * RPT, RPT)],
                    agg_out.at[c, pl.ds(s * RPT, RPT)])


_agg_call = pl.kernel(
    _agg_body,
    out_type=jax.ShapeDtypeStruct((NC, N, D), jnp.float32),
    mesh=_mesh,
    scratch_types=[
        pltpu.VMEM((CH,), jnp.int32),
        pltpu.VMEM((CH,), jnp.int32),
        pltpu.VMEM((REM,), jnp.int32),
        pltpu.VMEM((REM,), jnp.int32),
        pltpu.VMEM((CH, D), jnp.float32),
        pltpu.VMEM((125, D), jnp.float32),
        pltpu.VMEM_SHARED((N, D), jnp.float32),
        pltpu.SemaphoreType.DMA,
    ],
)


def _norm_body(p_ref, o_ref):
    x = p_ref[...]
    sdeg = jnp.sum(x[:NW], axis=0, keepdims=True)
    ddeg = jnp.sum(x[NW:], axis=0, keepdims=True)
    deg = jnp.concatenate([sdeg, ddeg], axis=0)
    o_ref[...] = lax.rsqrt(jnp.maximum(deg, 1.0))


_norm_call = pl.pallas_call(
    _norm_body,
    out_shape=jax.ShapeDtypeStruct((2, N), jnp.float32),
)

_RB = 400


def _mm_body(f_ref, ns_ref, w_ref, o_ref):
    o_ref[...] = jnp.dot(f_ref[...] * ns_ref[...], w_ref[...],
                         preferred_element_type=jnp.float32)


_mm_call = pl.pallas_call(
    _mm_body,
    grid=(N // _RB,),
    in_specs=[
        pl.BlockSpec((_RB, D), lambda i: (i, 0)),
        pl.BlockSpec((_RB, 1), lambda i: (i, 0)),
        pl.BlockSpec((D, D), lambda i: (0, 0)),
    ],
    out_specs=pl.BlockSpec((_RB, D), lambda i: (i, 0)),
    out_shape=jax.ShapeDtypeStruct((N, D), jnp.float32),
)


def _fin_body(a_ref, b_ref, nd_ref, bias_ref, o_ref):
    o_ref[...] = (a_ref[...] + b_ref[...]) * nd_ref[...] + bias_ref[...]


_fin_call = pl.pallas_call(
    _fin_body,
    grid=(N // _RB,),
    in_specs=[
        pl.BlockSpec((_RB, D), lambda i: (i, 0)),
        pl.BlockSpec((_RB, D), lambda i: (i, 0)),
        pl.BlockSpec((_RB, 1), lambda i: (i, 0)),
        pl.BlockSpec((1, D), lambda i: (0, 0)),
    ],
    out_specs=pl.BlockSpec((_RB, D), lambda i: (i, 0)),
    out_shape=jax.ShapeDtypeStruct((N, D), jnp.float32),
)


def kernel(feat, edge_index, weight, bias):
    src = edge_index[0]
    dst = edge_index[1]
    deg_part = _deg_call(src, dst)
    norms = _norm_call(deg_part.reshape(2 * NW, N))
    ns_col = norms[0].reshape(N, 1)
    nd_col = norms[1].reshape(N, 1)
    q = _mm_call(feat, ns_col, weight)
    agg = _agg_call(q, src, dst)
    return _fin_call(agg[0], agg[1], nd_col, bias.reshape(1, D))


# speedup vs baseline: 5.9305x; 5.9305x over previous
import functools

import jax
import jax.numpy as jnp
from jax import lax
from jax.experimental import pallas as pl
from jax.experimental.pallas import tpu as pltpu
from jax.experimental.pallas import tpu_sc as plsc

N = 10000
E = 320000
D = 128

NC = 2
NS = 16
NW = NC * NS
EPW = E // NW
CH = 128
NFULL = EPW // CH
REM = EPW - NFULL * CH
NPAD = 10240
RPT = NPAD // NS
V = 16

_mesh = plsc.VectorSubcoreMesh(core_axis_name="c", subcore_axis_name="s")
_sc_params = pltpu.CompilerParams(needs_layout_passes=False)


def _zero_vmem(ref, n):
    zeros = jnp.zeros((V,), jnp.float32)

    def _z(i, _):
        ref[pl.ds(i * V, V)] = zeros
        return 0

    lax.fori_loop(0, n // V, _z, 0)


def _deg_body(src_hbm, cnt_out, sidx, sidx_r, ones, zbuf, acc):
    c = lax.axis_index("c")
    s = lax.axis_index("s")
    w = s * NC + c

    _zero_vmem(ones, CH)
    one = jnp.ones((V,), jnp.float32)

    def _o(i, _):
        ones[pl.ds(i * V, V)] = one
        return 0

    lax.fori_loop(0, CH // V, _o, 0)
    _zero_vmem(zbuf, RPT)
    pltpu.sync_copy(zbuf, acc.at[pl.ds(s * RPT, RPT)])
    plsc.subcore_barrier()

    base_w = w * EPW

    def _chunk(j, _):
        base = base_w + j * CH
        pltpu.sync_copy(src_hbm.at[pl.ds(base, CH)], sidx)
        pltpu.sync_copy(ones, acc.at[sidx], add=True)
        return 0

    lax.fori_loop(0, NFULL, _chunk, 0)

    rbase = base_w + NFULL * CH
    pltpu.sync_copy(src_hbm.at[pl.ds(rbase, REM)], sidx_r)
    pltpu.sync_copy(ones.at[pl.ds(0, REM)], acc.at[sidx_r], add=True)

    plsc.subcore_barrier()
    pltpu.sync_copy(acc.at[pl.ds(s * RPT, RPT)],
                    cnt_out.at[c, pl.ds(s * RPT, RPT)])


_deg_call = pl.kernel(
    _deg_body,
    out_type=jax.ShapeDtypeStruct((NC, NPAD), jnp.float32),
    mesh=_mesh,
    scratch_types=[
        pltpu.VMEM((CH,), jnp.int32),
        pltpu.VMEM((REM,), jnp.int32),
        pltpu.VMEM((CH,), jnp.float32),
        pltpu.VMEM((RPT,), jnp.float32),
        pltpu.VMEM_SHARED((NPAD,), jnp.float32),
    ],
    compiler_params=_sc_params,
)


def _agg_body(q_hbm, src_hbm, dst_hbm, agg_out, cnt_out,
              sidx, didx, sidx_r, didx_r, rows, ones, zbuf, acc, acc_d, sem):
    c = lax.axis_index("c")
    s = lax.axis_index("s")
    w = s * NC + c

    zeros = jnp.zeros((V,), jnp.float32)

    def _z0(i, _):
        def _z1(j, _):
            zbuf[i, pl.ds(j * V, V)] = zeros
            return 0
        lax.fori_loop(0, D // V, _z1, 0)
        return 0

    lax.fori_loop(0, CH, _z0, 0)
    for k in range(RPT // CH):
        pltpu.sync_copy(zbuf, acc.at[pl.ds(s * RPT + k * CH, CH)])
        pltpu.sync_copy(zbuf.at[0], acc_d.at[pl.ds(s * RPT + k * CH, CH)])

    one = jnp.ones((V,), jnp.float32)

    def _o(i, _):
        ones[pl.ds(i * V, V)] = one
        return 0

    lax.fori_loop(0, CH // V, _o, 0)
    plsc.subcore_barrier()

    base_w = w * EPW

    def _chunk(j, _):
        base = base_w + j * CH
        pltpu.sync_copy(src_hbm.at[pl.ds(base, CH)], sidx)
        pltpu.sync_copy(dst_hbm.at[pl.ds(base, CH)], didx)
        pltpu.async_copy(q_hbm.at[sidx], rows, sem).wait()
        pltpu.sync_copy(rows, acc.at[didx], add=True)
        pltpu.sync_copy(ones, acc_d.at[didx], add=True)
        return 0

    lax.fori_loop(0, NFULL, _chunk, 0)

    rbase = base_w + NFULL * CH
    pltpu.sync_copy(src_hbm.at[pl.ds(rbase, REM)], sidx_r)
    pltpu.sync_copy(dst_hbm.at[pl.ds(rbase, REM)], didx_r)
    pltpu.async_copy(q_hbm.at[sidx_r], rows.at[pl.ds(0, REM)], sem).wait()
    pltpu.sync_copy(rows.at[pl.ds(0, REM)], acc.at[didx_r], add=True)
    pltpu.sync_copy(ones.at[pl.ds(0, REM)], acc_d.at[didx_r], add=True)

    plsc.subcore_barrier()
    pltpu.sync_copy(acc.at[pl.ds(s * RPT, RPT)],
                    agg_out.at[c, pl.ds(s * RPT, RPT)])
    pltpu.sync_copy(acc_d.at[pl.ds(s * RPT, RPT)],
                    cnt_out.at[c, pl.ds(s * RPT, RPT)])


_agg_call = pl.kernel(
    _agg_body,
    out_type=(
        jax.ShapeDtypeStruct((NC, NPAD, D), jnp.float32),
        jax.ShapeDtypeStruct((NC, NPAD), jnp.float32),
    ),
    mesh=_mesh,
    scratch_types=[
        pltpu.VMEM((CH,), jnp.int32),
        pltpu.VMEM((CH,), jnp.int32),
        pltpu.VMEM((REM,), jnp.int32),
        pltpu.VMEM((REM,), jnp.int32),
        pltpu.VMEM((CH, D), jnp.float32),
        pltpu.VMEM((CH,), jnp.float32),
        pltpu.VMEM((CH, D), jnp.float32),
        pltpu.VMEM_SHARED((NPAD, D), jnp.float32),
        pltpu.VMEM_SHARED((NPAD,), jnp.float32),
        pltpu.SemaphoreType.DMA,
    ],
    compiler_params=_sc_params,
)


def _norm_body(p_ref, o_ref):
    deg = jnp.sum(p_ref[...], axis=0, keepdims=True)
    o_ref[...] = 1.0 / jnp.sqrt(jnp.maximum(deg, 1.0))


_norm_call = pl.pallas_call(
    _norm_body,
    out_shape=jax.ShapeDtypeStruct((1, N), jnp.float32),
)

_RB = 400


def _mm_body(f_ref, ns_ref, w_ref, o_ref):
    o_ref[...] = jnp.dot(f_ref[...] * ns_ref[...], w_ref[...],
                         preferred_element_type=jnp.float32,
                         precision=lax.Precision.HIGHEST)


_mm_call = pl.pallas_call(
    _mm_body,
    grid=(N // _RB,),
    in_specs=[
        pl.BlockSpec((_RB, D), lambda i: (i, 0)),
        pl.BlockSpec((_RB, 1), lambda i: (i, 0)),
        pl.BlockSpec((D, D), lambda i: (0, 0)),
    ],
    out_specs=pl.BlockSpec((_RB, D), lambda i: (i, 0)),
    out_shape=jax.ShapeDtypeStruct((N, D), jnp.float32),
)


def _fin_body(a_ref, b_ref, c0_ref, c1_ref, bias_ref, o_ref):
    nd = 1.0 / jnp.sqrt(jnp.maximum(c0_ref[...] + c1_ref[...], 1.0))
    o_ref[...] = (a_ref[...] + b_ref[...]) * nd + bias_ref[...]


_fin_call = pl.pallas_call(
    _fin_body,
    grid=(N // _RB,),
    in_specs=[
        pl.BlockSpec((_RB, D), lambda i: (i, 0)),
        pl.BlockSpec((_RB, D), lambda i: (i, 0)),
        pl.BlockSpec((_RB, 1), lambda i: (i, 0)),
        pl.BlockSpec((_RB, 1), lambda i: (i, 0)),
        pl.BlockSpec((1, D), lambda i: (0, 0)),
    ],
    out_specs=pl.BlockSpec((_RB, D), lambda i: (i, 0)),
    out_shape=jax.ShapeDtypeStruct((N, D), jnp.float32),
)


def kernel(feat, edge_index, weight, bias):
    src = edge_index[0]
    dst = edge_index[1]
    scnt = _deg_call(src)
    ns_col = _norm_call(scnt[:, :N]).reshape(N, 1)
    q = _mm_call(feat, ns_col, weight)
    agg, dcnt = _agg_call(q, src, dst)
    return _fin_call(agg[0, :N], agg[1, :N],
                     dcnt[0, :N].reshape(N, 1), dcnt[1, :N].reshape(N, 1),
                     bias.reshape(1, D))


# speedup vs baseline: 8.6610x; 1.4604x over previous
import functools

import jax
import jax.numpy as jnp
from jax import lax
from jax.experimental import pallas as pl
from jax.experimental.pallas import tpu as pltpu
from jax.experimental.pallas import tpu_sc as plsc

N = 10000
E = 320000
D = 128

NC = 2
NS = 16
NW = NC * NS
EPW = E // NW
CH = 128
NFULL = EPW // CH
REM = EPW - NFULL * CH
NPAD = 10240
RPT = NPAD // NS
V = 16

_mesh = plsc.VectorSubcoreMesh(core_axis_name="c", subcore_axis_name="s")
_sc_params = pltpu.CompilerParams(needs_layout_passes=False)


def _zero_vmem(ref, n):
    zeros = jnp.zeros((V,), jnp.float32)

    def _z(i, _):
        ref[pl.ds(i * V, V)] = zeros
        return 0

    lax.fori_loop(0, n // V, _z, 0)


def _deg_body(src_hbm, cnt_out, sidx, sidx_r, ones, zbuf, acc):
    c = lax.axis_index("c")
    s = lax.axis_index("s")
    w = s * NC + c

    _zero_vmem(ones, CH)
    one = jnp.ones((V,), jnp.float32)

    def _o(i, _):
        ones[pl.ds(i * V, V)] = one
        return 0

    lax.fori_loop(0, CH // V, _o, 0)
    _zero_vmem(zbuf, RPT)
    pltpu.sync_copy(zbuf, acc.at[pl.ds(s * RPT, RPT)])
    plsc.subcore_barrier()

    base_w = w * EPW

    def _chunk(j, _):
        base = base_w + j * CH
        pltpu.sync_copy(src_hbm.at[pl.ds(base, CH)], sidx)
        pltpu.sync_copy(ones, acc.at[sidx], add=True)
        return 0

    lax.fori_loop(0, NFULL, _chunk, 0)

    rbase = base_w + NFULL * CH
    pltpu.sync_copy(src_hbm.at[pl.ds(rbase, REM)], sidx_r)
    pltpu.sync_copy(ones.at[pl.ds(0, REM)], acc.at[sidx_r], add=True)

    plsc.subcore_barrier()
    pltpu.sync_copy(acc.at[pl.ds(s * RPT, RPT)],
                    cnt_out.at[c, pl.ds(s * RPT, RPT)])


_deg_call = pl.kernel(
    _deg_body,
    out_type=jax.ShapeDtypeStruct((NC, NPAD), jnp.float32),
    mesh=_mesh,
    scratch_types=[
        pltpu.VMEM((CH,), jnp.int32),
        pltpu.VMEM((REM,), jnp.int32),
        pltpu.VMEM((CH,), jnp.float32),
        pltpu.VMEM((RPT,), jnp.float32),
        pltpu.VMEM_SHARED((NPAD,), jnp.float32),
    ],
    compiler_params=_sc_params,
)


NCHUNK = E // CH
NJ = -(-NCHUNK // NW)
NBUF = 2


def _agg_body(q_hbm, ei3_hbm, agg_out, cnt_out,
              idxb, rows, ones, acc, acc_d, isem, gsem):
    c = lax.axis_index("c")
    s = lax.axis_index("s")
    w = s * NC + c

    zeros = jnp.zeros((V,), jnp.float32)

    def _z0(i, _):
        def _z1(j, _):
            rows[0, i, pl.ds(j * V, V)] = zeros
            return 0
        lax.fori_loop(0, D // V, _z1, 0)
        return 0

    lax.fori_loop(0, CH, _z0, 0)
    for k in range(RPT // CH):
        pltpu.sync_copy(rows.at[0], acc.at[pl.ds(s * RPT + k * CH, CH)])
        pltpu.sync_copy(rows.at[0, 0], acc_d.at[pl.ds(s * RPT + k * CH, CH)])

    one = jnp.ones((V,), jnp.float32)

    def _o(i, _):
        ones[pl.ds(i * V, V)] = one
        return 0

    lax.fori_loop(0, CH // V, _o, 0)
    plsc.subcore_barrier()

    def _cid(j):
        return w + j * NW

    def _start_idx(j, b):
        pltpu.async_copy(ei3_hbm.at[:, _cid(j)], idxb.at[b], isem.at[b])

    def _wait_idx(j, b):
        pltpu.make_async_copy(ei3_hbm.at[:, _cid(j)], idxb.at[b],
                              isem.at[b]).wait()

    def _start_gather(b):
        pltpu.async_copy(q_hbm.at[idxb.at[b, 0]], rows.at[b], gsem.at[b])

    def _wait_gather(b):
        pltpu.make_async_copy(q_hbm.at[idxb.at[b, 0]], rows.at[b],
                              gsem.at[b]).wait()

    for t in range(NBUF):
        @pl.when(_cid(t) < NCHUNK)
        def _():
            _start_idx(t, t)

    @pl.when(_cid(0) < NCHUNK)
    def _():
        _wait_idx(0, 0)
        _start_gather(0)

    def _step(j, _):
        b = lax.rem(j, NBUF)
        nb = lax.rem(j + 1, NBUF)

        @pl.when(_cid(j + 1) < NCHUNK)
        def _():
            _wait_idx(j + 1, nb)
            _start_gather(nb)

        @pl.when(_cid(j) < NCHUNK)
        def _():
            _wait_gather(b)
            pltpu.sync_copy(rows.at[b], acc.at[idxb.at[b, 1]], add=True)
            pltpu.sync_copy(ones, acc_d.at[idxb.at[b, 1]], add=True)

        @pl.when(_cid(j + NBUF) < NCHUNK)
        def _():
            _start_idx(j + NBUF, b)

        return 0

    lax.fori_loop(0, NJ, _step, 0)

    plsc.subcore_barrier()
    pltpu.sync_copy(acc.at[pl.ds(s * RPT, RPT)],
                    agg_out.at[c, pl.ds(s * RPT, RPT)])
    pltpu.sync_copy(acc_d.at[pl.ds(s * RPT, RPT)],
                    cnt_out.at[c, pl.ds(s * RPT, RPT)])


_agg_call = pl.kernel(
    _agg_body,
    out_type=(
        jax.ShapeDtypeStruct((NC, NPAD, D), jnp.float32),
        jax.ShapeDtypeStruct((NC, NPAD), jnp.float32),
    ),
    mesh=_mesh,
    scratch_types=[
        pltpu.VMEM((NBUF, 2, CH), jnp.int32),
        pltpu.VMEM((NBUF, CH, D), jnp.float32),
        pltpu.VMEM((CH,), jnp.float32),
        pltpu.VMEM_SHARED((NPAD, D), jnp.float32),
        pltpu.VMEM_SHARED((NPAD,), jnp.float32),
        pltpu.SemaphoreType.DMA((NBUF,)),
        pltpu.SemaphoreType.DMA((NBUF,)),
    ],
    compiler_params=_sc_params,
)


_RB = 400


def _mm_body(f_ref, c0_ref, c1_ref, w_ref, o_ref):
    ns = 1.0 / jnp.sqrt(jnp.maximum(c0_ref[...] + c1_ref[...], 1.0))
    o_ref[...] = jnp.dot(f_ref[...] * ns, w_ref[...],
                         preferred_element_type=jnp.float32,
                         precision=lax.Precision.HIGHEST)


_mm_call = pl.pallas_call(
    _mm_body,
    grid=(N // _RB,),
    in_specs=[
        pl.BlockSpec((_RB, D), lambda i: (i, 0)),
        pl.BlockSpec((_RB, 1), lambda i: (i, 0)),
        pl.BlockSpec((_RB, 1), lambda i: (i, 0)),
        pl.BlockSpec((D, D), lambda i: (0, 0)),
    ],
    out_specs=pl.BlockSpec((_RB, D), lambda i: (i, 0)),
    out_shape=jax.ShapeDtypeStruct((N, D), jnp.float32),
)


def _fin_body(a_ref, b_ref, c0_ref, c1_ref, bias_ref, o_ref):
    nd = 1.0 / jnp.sqrt(jnp.maximum(c0_ref[...] + c1_ref[...], 1.0))
    o_ref[...] = (a_ref[...] + b_ref[...]) * nd + bias_ref[...]


_fin_call = pl.pallas_call(
    _fin_body,
    grid=(N // _RB,),
    in_specs=[
        pl.BlockSpec((_RB, D), lambda i: (i, 0)),
        pl.BlockSpec((_RB, D), lambda i: (i, 0)),
        pl.BlockSpec((_RB, 1), lambda i: (i, 0)),
        pl.BlockSpec((_RB, 1), lambda i: (i, 0)),
        pl.BlockSpec((1, D), lambda i: (0, 0)),
    ],
    out_specs=pl.BlockSpec((_RB, D), lambda i: (i, 0)),
    out_shape=jax.ShapeDtypeStruct((N, D), jnp.float32),
)


def kernel(feat, edge_index, weight, bias):
    src = edge_index[0]
    ei3 = edge_index.reshape(2, NCHUNK, CH)
    scnt = _deg_call(src)
    c0 = scnt[0, :N].reshape(N, 1)
    c1 = scnt[1, :N].reshape(N, 1)
    q = _mm_call(feat, c0, c1, weight)
    agg, dcnt = _agg_call(q, ei3)
    return _fin_call(agg[0, :N], agg[1, :N],
                     dcnt[0, :N].reshape(N, 1), dcnt[1, :N].reshape(N, 1),
                     bias.reshape(1, D))


# speedup vs baseline: 10.7199x; 1.2377x over previous
import functools

import jax
import jax.numpy as jnp
from jax import lax
from jax.experimental import pallas as pl
from jax.experimental.pallas import tpu as pltpu
from jax.experimental.pallas import tpu_sc as plsc

N = 10000
E = 320000
D = 128

NC = 2
NS = 16
NW = NC * NS
EPW = E // NW
CH = 128
NFULL = EPW // CH
REM = EPW - NFULL * CH
NPAD = 10240
RPT = NPAD // NS
V = 16

_mesh = plsc.VectorSubcoreMesh(core_axis_name="c", subcore_axis_name="s")
_sc_params = pltpu.CompilerParams(needs_layout_passes=False)


def _zero_vmem(ref, n):
    zeros = jnp.zeros((V,), jnp.float32)

    def _z(i, _):
        ref[pl.ds(i * V, V)] = zeros
        return 0

    lax.fori_loop(0, n // V, _z, 0)


NCHUNK = E // CH
NJ = -(-NCHUNK // NW)
DEG_NBUF = 4


def _deg_body(ei3_hbm, cnt_out, idxb, ones, zbuf, acc, isem):
    c = lax.axis_index("c")
    s = lax.axis_index("s")
    w = s * NC + c

    one = jnp.ones((V,), jnp.float32)

    def _o(i, _):
        ones[pl.ds(i * V, V)] = one
        return 0

    lax.fori_loop(0, CH // V, _o, 0)
    _zero_vmem(zbuf, RPT)
    pltpu.sync_copy(zbuf, acc.at[pl.ds(s * RPT, RPT)])
    plsc.subcore_barrier()

    def _cid(j):
        return w + j * NW

    def _start_idx(j, b):
        pltpu.async_copy(ei3_hbm.at[0, _cid(j)], idxb.at[b], isem.at[b])

    def _wait_idx(j, b):
        pltpu.make_async_copy(ei3_hbm.at[0, _cid(j)], idxb.at[b],
                              isem.at[b]).wait()

    for t in range(DEG_NBUF):
        @pl.when(_cid(t) < NCHUNK)
        def _():
            _start_idx(t, t)

    def _step(j, _):
        b = lax.rem(j, DEG_NBUF)

        @pl.when(_cid(j) < NCHUNK)
        def _():
            _wait_idx(j, b)
            pltpu.sync_copy(ones, acc.at[idxb.at[b]], add=True)

        @pl.when(_cid(j + DEG_NBUF) < NCHUNK)
        def _():
            _start_idx(j + DEG_NBUF, b)

        return 0

    lax.fori_loop(0, NJ, _step, 0)

    plsc.subcore_barrier()
    pltpu.sync_copy(acc.at[pl.ds(s * RPT, RPT)],
                    cnt_out.at[c, pl.ds(s * RPT, RPT)])


_deg_call = pl.kernel(
    _deg_body,
    out_type=jax.ShapeDtypeStruct((NC, NPAD), jnp.float32),
    mesh=_mesh,
    scratch_types=[
        pltpu.VMEM((DEG_NBUF, CH), jnp.int32),
        pltpu.VMEM((CH,), jnp.float32),
        pltpu.VMEM((RPT,), jnp.float32),
        pltpu.VMEM_SHARED((NPAD,), jnp.float32),
        pltpu.SemaphoreType.DMA((DEG_NBUF,)),
    ],
    compiler_params=_sc_params,
)


NBUF = 2


def _agg_body(q_hbm, ei3_hbm, agg_out, cnt_out,
              idxb, rows, ones, acc, acc_d, isem, gsem):
    c = lax.axis_index("c")
    s = lax.axis_index("s")
    w = s * NC + c

    zeros = jnp.zeros((V,), jnp.float32)

    def _z0(i, _):
        def _z1(j, _):
            rows[0, i, pl.ds(j * V, V)] = zeros
            return 0
        lax.fori_loop(0, D // V, _z1, 0)
        return 0

    lax.fori_loop(0, CH, _z0, 0)
    for k in range(RPT // CH):
        pltpu.sync_copy(rows.at[0], acc.at[pl.ds(s * RPT + k * CH, CH)])
        pltpu.sync_copy(rows.at[0, 0], acc_d.at[pl.ds(s * RPT + k * CH, CH)])

    one = jnp.ones((V,), jnp.float32)

    def _o(i, _):
        ones[pl.ds(i * V, V)] = one
        return 0

    lax.fori_loop(0, CH // V, _o, 0)
    plsc.subcore_barrier()

    def _cid(j):
        return w + j * NW

    def _start_idx(j, b):
        pltpu.async_copy(ei3_hbm.at[:, _cid(j)], idxb.at[b], isem.at[b])

    def _wait_idx(j, b):
        pltpu.make_async_copy(ei3_hbm.at[:, _cid(j)], idxb.at[b],
                              isem.at[b]).wait()

    def _start_gather(b):
        pltpu.async_copy(q_hbm.at[idxb.at[b, 0]], rows.at[b], gsem.at[b])

    def _wait_gather(b):
        pltpu.make_async_copy(q_hbm.at[idxb.at[b, 0]], rows.at[b],
                              gsem.at[b]).wait()

    for t in range(NBUF):
        @pl.when(_cid(t) < NCHUNK)
        def _():
            _start_idx(t, t)

    @pl.when(_cid(0) < NCHUNK)
    def _():
        _wait_idx(0, 0)
        _start_gather(0)

    def _step(j, _):
        b = lax.rem(j, NBUF)
        nb = lax.rem(j + 1, NBUF)

        @pl.when(_cid(j + 1) < NCHUNK)
        def _():
            _wait_idx(j + 1, nb)
            _start_gather(nb)

        @pl.when(_cid(j) < NCHUNK)
        def _():
            _wait_gather(b)
            pltpu.sync_copy(rows.at[b], acc.at[idxb.at[b, 1]], add=True)
            pltpu.sync_copy(ones, acc_d.at[idxb.at[b, 1]], add=True)

        @pl.when(_cid(j + NBUF) < NCHUNK)
        def _():
            _start_idx(j + NBUF, b)

        return 0

    lax.fori_loop(0, NJ, _step, 0)

    plsc.subcore_barrier()
    pltpu.sync_copy(acc.at[pl.ds(s * RPT, RPT)],
                    agg_out.at[c, pl.ds(s * RPT, RPT)])
    pltpu.sync_copy(acc_d.at[pl.ds(s * RPT, RPT)],
                    cnt_out.at[c, pl.ds(s * RPT, RPT)])


_agg_call = pl.kernel(
    _agg_body,
    out_type=(
        jax.ShapeDtypeStruct((NC, NPAD, D), jnp.float32),
        jax.ShapeDtypeStruct((NC, NPAD), jnp.float32),
    ),
    mesh=_mesh,
    scratch_types=[
        pltpu.VMEM((NBUF, 2, CH), jnp.int32),
        pltpu.VMEM((NBUF, CH, D), jnp.float32),
        pltpu.VMEM((CH,), jnp.float32),
        pltpu.VMEM_SHARED((NPAD, D), jnp.float32),
        pltpu.VMEM_SHARED((NPAD,), jnp.float32),
        pltpu.SemaphoreType.DMA((NBUF,)),
        pltpu.SemaphoreType.DMA((NBUF,)),
    ],
    compiler_params=_sc_params,
)


_RB = 400


def _mm_body(f_ref, c0_ref, c1_ref, w_ref, o_ref):
    ns = 1.0 / jnp.sqrt(jnp.maximum(c0_ref[0] + c1_ref[0], 1.0))
    o_ref[...] = jnp.dot(f_ref[...] * ns, w_ref[...],
                         preferred_element_type=jnp.float32,
                         precision=lax.Precision.HIGHEST)


_mm_call = pl.pallas_call(
    _mm_body,
    grid=(N // _RB,),
    in_specs=[
        pl.BlockSpec((_RB, D), lambda i: (i, 0)),
        pl.BlockSpec((1, _RB, 1), lambda i: (0, i, 0)),
        pl.BlockSpec((1, _RB, 1), lambda i: (1, i, 0)),
        pl.BlockSpec((D, D), lambda i: (0, 0)),
    ],
    out_specs=pl.BlockSpec((_RB, D), lambda i: (i, 0)),
    out_shape=jax.ShapeDtypeStruct((N, D), jnp.float32),
)


def _fin_body(a_ref, b_ref, c0_ref, c1_ref, bias_ref, o_ref):
    nd = 1.0 / jnp.sqrt(jnp.maximum(c0_ref[0] + c1_ref[0], 1.0))
    o_ref[...] = (a_ref[0] + b_ref[0]) * nd + bias_ref[...]


_fin_call = pl.pallas_call(
    _fin_body,
    grid=(N // _RB,),
    in_specs=[
        pl.BlockSpec((1, _RB, D), lambda i: (0, i, 0)),
        pl.BlockSpec((1, _RB, D), lambda i: (1, i, 0)),
        pl.BlockSpec((1, _RB, 1), lambda i: (0, i, 0)),
        pl.BlockSpec((1, _RB, 1), lambda i: (1, i, 0)),
        pl.BlockSpec((1, D), lambda i: (0, 0)),
    ],
    out_specs=pl.BlockSpec((_RB, D), lambda i: (i, 0)),
    out_shape=jax.ShapeDtypeStruct((N, D), jnp.float32),
)


def kernel(feat, edge_index, weight, bias):
    ei3 = edge_index.reshape(2, NCHUNK, CH)
    scnt = _deg_call(ei3)
    q = _mm_call(feat, scnt.reshape(NC, NPAD, 1), scnt.reshape(NC, NPAD, 1),
                 weight)
    agg, dcnt = _agg_call(q, ei3)
    return _fin_call(agg, agg, dcnt.reshape(NC, NPAD, 1),
                     dcnt.reshape(NC, NPAD, 1), bias.reshape(1, D))


# speedup vs baseline: 11.6644x; 1.0881x over previous
import functools

import jax
import jax.numpy as jnp
from jax import lax
from jax.experimental import pallas as pl
from jax.experimental.pallas import tpu as pltpu
from jax.experimental.pallas import tpu_sc as plsc

N = 10000
E = 320000
D = 128

NC = 2
NS = 16
NW = NC * NS
EPW = E // NW
CH = 128
NFULL = EPW // CH
REM = EPW - NFULL * CH
NPAD = 10240
RPT = NPAD // NS
V = 16

_mesh = plsc.VectorSubcoreMesh(core_axis_name="c", subcore_axis_name="s")
_sc_params = pltpu.CompilerParams(needs_layout_passes=False)


def _zero_vmem(ref, n):
    zeros = jnp.zeros((V,), jnp.float32)

    def _z(i, _):
        ref[pl.ds(i * V, V)] = zeros
        return 0

    lax.fori_loop(0, n // V, _z, 0)


NCHUNK = E // CH
NJ = -(-NCHUNK // NW)
DEG_NBUF = 4


def _deg_body(ei3_hbm, cnt_out, idxb, ones, zbuf, acc, isem):
    c = lax.axis_index("c")
    s = lax.axis_index("s")
    w = s * NC + c

    one = jnp.ones((V,), jnp.float32)

    def _o(i, _):
        ones[pl.ds(i * V, V)] = one
        return 0

    lax.fori_loop(0, CH // V, _o, 0)
    _zero_vmem(zbuf, RPT)
    pltpu.sync_copy(zbuf, acc.at[pl.ds(s * RPT, RPT)])
    plsc.subcore_barrier()

    def _cid(j):
        return w + j * NW

    def _start_idx(j, b):
        pltpu.async_copy(ei3_hbm.at[0, _cid(j)], idxb.at[b], isem.at[b])

    def _wait_idx(j, b):
        pltpu.make_async_copy(ei3_hbm.at[0, _cid(j)], idxb.at[b],
                              isem.at[b]).wait()

    for t in range(DEG_NBUF):
        @pl.when(_cid(t) < NCHUNK)
        def _():
            _start_idx(t, t)

    def _step(j, _):
        b = lax.rem(j, DEG_NBUF)

        @pl.when(_cid(j) < NCHUNK)
        def _():
            _wait_idx(j, b)
            pltpu.sync_copy(ones, acc.at[idxb.at[b]], add=True)

        @pl.when(_cid(j + DEG_NBUF) < NCHUNK)
        def _():
            _start_idx(j + DEG_NBUF, b)

        return 0

    lax.fori_loop(0, NJ, _step, 0)

    plsc.subcore_barrier()
    pltpu.sync_copy(acc.at[pl.ds(s * RPT, RPT)],
                    cnt_out.at[c, pl.ds(s * RPT, RPT)])


_deg_call = pl.kernel(
    _deg_body,
    out_type=jax.ShapeDtypeStruct((NC, NPAD), jnp.float32),
    mesh=_mesh,
    scratch_types=[
        pltpu.VMEM((DEG_NBUF, CH), jnp.int32),
        pltpu.VMEM((CH,), jnp.float32),
        pltpu.VMEM((RPT,), jnp.float32),
        pltpu.VMEM_SHARED((NPAD,), jnp.float32),
        pltpu.SemaphoreType.DMA((DEG_NBUF,)),
    ],
    compiler_params=_sc_params,
)


NBUF = 2
NIB = 3


def _agg_body(q_hbm, ei3_hbm, agg_out, cnt_out,
              idxb, rows, ones, acc, acc_d, isem, gsem, ssem, osem):
    c = lax.axis_index("c")
    s = lax.axis_index("s")
    w = s * NC + c

    zeros = jnp.zeros((V,), jnp.float32)

    def _z0(i, _):
        def _z1(j, _):
            rows[0, i, pl.ds(j * V, V)] = zeros
            return 0
        lax.fori_loop(0, D // V, _z1, 0)
        return 0

    lax.fori_loop(0, CH, _z0, 0)
    for k in range(RPT // CH):
        pltpu.sync_copy(rows.at[0], acc.at[pl.ds(s * RPT + k * CH, CH)])
        pltpu.sync_copy(rows.at[0, 0], acc_d.at[pl.ds(s * RPT + k * CH, CH)])

    one = jnp.ones((V,), jnp.float32)

    def _o(i, _):
        ones[pl.ds(i * V, V)] = one
        return 0

    lax.fori_loop(0, CH // V, _o, 0)
    plsc.subcore_barrier()

    def _cid(j):
        return w + j * NW

    def _start_idx(j, bi):
        pltpu.async_copy(ei3_hbm.at[:, _cid(j)], idxb.at[bi], isem.at[bi])

    def _wait_idx(j, bi):
        pltpu.make_async_copy(ei3_hbm.at[:, _cid(j)], idxb.at[bi],
                              isem.at[bi]).wait()

    def _start_gather(bi, br):
        pltpu.async_copy(q_hbm.at[idxb.at[bi, 0]], rows.at[br], gsem.at[br])

    def _wait_gather(bi, br):
        pltpu.make_async_copy(q_hbm.at[idxb.at[bi, 0]], rows.at[br],
                              gsem.at[br]).wait()

    def _start_scat(bi, br):
        pltpu.async_copy(rows.at[br], acc.at[idxb.at[bi, 1]], ssem.at[br],
                         add=True)
        pltpu.async_copy(ones, acc_d.at[idxb.at[bi, 1]], osem.at[br],
                         add=True)

    def _wait_scat(bi, br):
        pltpu.make_async_copy(rows.at[br], acc.at[idxb.at[bi, 1]],
                              ssem.at[br]).wait()
        pltpu.make_async_copy(ones, acc_d.at[idxb.at[bi, 1]],
                              osem.at[br]).wait()

    for t in range(NIB - 1):
        @pl.when(_cid(t) < NCHUNK)
        def _():
            _start_idx(t, t)

    @pl.when(_cid(0) < NCHUNK)
    def _():
        _wait_idx(0, 0)
        _start_gather(0, 0)

    def _step(j, _):
        br = lax.rem(j, NBUF)
        nbr = lax.rem(j + 1, NBUF)
        bi = lax.rem(j, NIB)
        nbi = lax.rem(j + 1, NIB)
        pbi = lax.rem(j + NIB - 1, NIB)

        @pl.when(_cid(j + 1) < NCHUNK)
        def _():
            @pl.when(j >= 1)
            def _():
                _wait_scat(pbi, nbr)
            _wait_idx(j + 1, nbi)
            _start_gather(nbi, nbr)

        @pl.when(_cid(j) < NCHUNK)
        def _():
            _wait_gather(bi, br)
            _start_scat(bi, br)

        @pl.when(_cid(j + 2) < NCHUNK)
        def _():
            _start_idx(j + 2, pbi)

        return 0

    lax.fori_loop(0, NJ, _step, 0)

    def _drain(j, _):
        @pl.when(jnp.logical_and(_cid(j) < NCHUNK,
                                 _cid(j + 2) >= NCHUNK))
        def _():
            _wait_scat(lax.rem(j, NIB), lax.rem(j, NBUF))
        return 0

    lax.fori_loop(NJ - 3, NJ, _drain, 0)

    plsc.subcore_barrier()
    pltpu.sync_copy(acc.at[pl.ds(s * RPT, RPT)],
                    agg_out.at[c, pl.ds(s * RPT, RPT)])
    pltpu.sync_copy(acc_d.at[pl.ds(s * RPT, RPT)],
                    cnt_out.at[c, pl.ds(s * RPT, RPT)])


_agg_call = pl.kernel(
    _agg_body,
    out_type=(
        jax.ShapeDtypeStruct((NC, NPAD, D), jnp.float32),
        jax.ShapeDtypeStruct((NC, NPAD), jnp.float32),
    ),
    mesh=_mesh,
    scratch_types=[
        pltpu.VMEM((NIB, 2, CH), jnp.int32),
        pltpu.VMEM((NBUF, CH, D), jnp.float32),
        pltpu.VMEM((CH,), jnp.float32),
        pltpu.VMEM_SHARED((NPAD, D), jnp.float32),
        pltpu.VMEM_SHARED((NPAD,), jnp.float32),
        pltpu.SemaphoreType.DMA((NIB,)),
        pltpu.SemaphoreType.DMA((NBUF,)),
        pltpu.SemaphoreType.DMA((NBUF,)),
        pltpu.SemaphoreType.DMA((NBUF,)),
    ],
    compiler_params=_sc_params,
)


_RB = 400


def _mm_body(f_ref, c0_ref, c1_ref, w_ref, o_ref):
    ns = 1.0 / jnp.sqrt(jnp.maximum(c0_ref[0] + c1_ref[0], 1.0))
    o_ref[...] = jnp.dot(f_ref[...] * ns, w_ref[...],
                         preferred_element_type=jnp.float32,
                         precision=lax.Precision.HIGHEST)


_mm_call = pl.pallas_call(
    _mm_body,
    grid=(N // _RB,),
    in_specs=[
        pl.BlockSpec((_RB, D), lambda i: (i, 0)),
        pl.BlockSpec((1, _RB, 1), lambda i: (0, i, 0)),
        pl.BlockSpec((1, _RB, 1), lambda i: (1, i, 0)),
        pl.BlockSpec((D, D), lambda i: (0, 0)),
    ],
    out_specs=pl.BlockSpec((_RB, D), lambda i: (i, 0)),
    out_shape=jax.ShapeDtypeStruct((N, D), jnp.float32),
)


def _fin_body(a_ref, b_ref, c0_ref, c1_ref, bias_ref, o_ref):
    nd = 1.0 / jnp.sqrt(jnp.maximum(c0_ref[0] + c1_ref[0], 1.0))
    o_ref[...] = (a_ref[0] + b_ref[0]) * nd + bias_ref[...]


_fin_call = pl.pallas_call(
    _fin_body,
    grid=(N // _RB,),
    in_specs=[
        pl.BlockSpec((1, _RB, D), lambda i: (0, i, 0)),
        pl.BlockSpec((1, _RB, D), lambda i: (1, i, 0)),
        pl.BlockSpec((1, _RB, 1), lambda i: (0, i, 0)),
        pl.BlockSpec((1, _RB, 1), lambda i: (1, i, 0)),
        pl.BlockSpec((1, D), lambda i: (0, 0)),
    ],
    out_specs=pl.BlockSpec((_RB, D), lambda i: (i, 0)),
    out_shape=jax.ShapeDtypeStruct((N, D), jnp.float32),
)


def kernel(feat, edge_index, weight, bias):
    ei3 = edge_index.reshape(2, NCHUNK, CH)
    scnt = _deg_call(ei3)
    q = _mm_call(feat, scnt.reshape(NC, NPAD, 1), scnt.reshape(NC, NPAD, 1),
                 weight)
    agg, dcnt = _agg_call(q, ei3)
    return _fin_call(agg, agg, dcnt.reshape(NC, NPAD, 1),
                     dcnt.reshape(NC, NPAD, 1), bias.reshape(1, D))


# speedup vs baseline: 11.8371x; 1.0148x over previous
import functools

import jax
import jax.numpy as jnp
from jax import lax
from jax.experimental import pallas as pl
from jax.experimental.pallas import tpu as pltpu
from jax.experimental.pallas import tpu_sc as plsc

N = 10000
E = 320000
D = 128

NC = 2
NS = 16
NW = NC * NS
EPW = E // NW
CH = 128
NFULL = EPW // CH
REM = EPW - NFULL * CH
NPAD = 10240
RPT = NPAD // NS
V = 16

_mesh = plsc.VectorSubcoreMesh(core_axis_name="c", subcore_axis_name="s")
_sc_params = pltpu.CompilerParams(needs_layout_passes=False)


def _zero_vmem(ref, n):
    zeros = jnp.zeros((V,), jnp.float32)

    def _z(i, _):
        ref[pl.ds(i * V, V)] = zeros
        return 0

    lax.fori_loop(0, n // V, _z, 0)


NCHUNK = E // CH
NJ = -(-NCHUNK // NW)
DEG_NBUF = 4


def _deg_body(ei3_hbm, scnt_out, dcnt_out, idxb, ones, zbuf, acc, acc_d, isem):
    c = lax.axis_index("c")
    s = lax.axis_index("s")
    w = s * NC + c

    one = jnp.ones((V,), jnp.float32)

    def _o(i, _):
        ones[pl.ds(i * V, V)] = one
        return 0

    lax.fori_loop(0, CH // V, _o, 0)
    _zero_vmem(zbuf, RPT)
    pltpu.sync_copy(zbuf, acc.at[pl.ds(s * RPT, RPT)])
    pltpu.sync_copy(zbuf, acc_d.at[pl.ds(s * RPT, RPT)])
    plsc.subcore_barrier()

    def _cid(j):
        return w + j * NW

    def _start_idx(j, b):
        pltpu.async_copy(ei3_hbm.at[:, _cid(j)], idxb.at[b], isem.at[b])

    def _wait_idx(j, b):
        pltpu.make_async_copy(ei3_hbm.at[:, _cid(j)], idxb.at[b],
                              isem.at[b]).wait()

    for t in range(DEG_NBUF):
        @pl.when(_cid(t) < NCHUNK)
        def _():
            _start_idx(t, t)

    def _step(j, _):
        b = lax.rem(j, DEG_NBUF)

        @pl.when(_cid(j) < NCHUNK)
        def _():
            _wait_idx(j, b)
            pltpu.sync_copy(ones, acc.at[idxb.at[b, 0]], add=True)
            pltpu.sync_copy(ones, acc_d.at[idxb.at[b, 1]], add=True)

        @pl.when(_cid(j + DEG_NBUF) < NCHUNK)
        def _():
            _start_idx(j + DEG_NBUF, b)

        return 0

    lax.fori_loop(0, NJ, _step, 0)

    plsc.subcore_barrier()
    pltpu.sync_copy(acc.at[pl.ds(s * RPT, RPT)],
                    scnt_out.at[c, pl.ds(s * RPT, RPT)])
    pltpu.sync_copy(acc_d.at[pl.ds(s * RPT, RPT)],
                    dcnt_out.at[c, pl.ds(s * RPT, RPT)])


_deg_call = pl.kernel(
    _deg_body,
    out_type=(
        jax.ShapeDtypeStruct((NC, NPAD), jnp.float32),
        jax.ShapeDtypeStruct((NC, NPAD), jnp.float32),
    ),
    mesh=_mesh,
    scratch_types=[
        pltpu.VMEM((DEG_NBUF, 2, CH), jnp.int32),
        pltpu.VMEM((CH,), jnp.float32),
        pltpu.VMEM((RPT,), jnp.float32),
        pltpu.VMEM_SHARED((NPAD,), jnp.float32),
        pltpu.VMEM_SHARED((NPAD,), jnp.float32),
        pltpu.SemaphoreType.DMA((DEG_NBUF,)),
    ],
    compiler_params=_sc_params,
)


NBUF = 2
NIB = 3


def _agg_body(q_hbm, ei3_hbm, agg_out,
              idxb, rows, acc, isem, gsem, ssem):
    c = lax.axis_index("c")
    s = lax.axis_index("s")
    w = s * NC + c

    zeros = jnp.zeros((V,), jnp.float32)

    def _z0(i, _):
        def _z1(j, _):
            rows[0, i, pl.ds(j * V, V)] = zeros
            return 0
        lax.fori_loop(0, D // V, _z1, 0)
        return 0

    lax.fori_loop(0, CH, _z0, 0)
    for k in range(RPT // CH):
        pltpu.sync_copy(rows.at[0], acc.at[pl.ds(s * RPT + k * CH, CH)])
    plsc.subcore_barrier()

    def _cid(j):
        return w + j * NW

    def _start_idx(j, bi):
        pltpu.async_copy(ei3_hbm.at[:, _cid(j)], idxb.at[bi], isem.at[bi])

    def _wait_idx(j, bi):
        pltpu.make_async_copy(ei3_hbm.at[:, _cid(j)], idxb.at[bi],
                              isem.at[bi]).wait()

    def _start_gather(bi, br):
        pltpu.async_copy(q_hbm.at[idxb.at[bi, 0]], rows.at[br], gsem.at[br])

    def _wait_gather(bi, br):
        pltpu.make_async_copy(q_hbm.at[idxb.at[bi, 0]], rows.at[br],
                              gsem.at[br]).wait()

    def _start_scat(bi, br):
        pltpu.async_copy(rows.at[br], acc.at[idxb.at[bi, 1]], ssem.at[br],
                         add=True)

    def _wait_scat(bi, br):
        pltpu.make_async_copy(rows.at[br], acc.at[idxb.at[bi, 1]],
                              ssem.at[br]).wait()

    for t in range(NIB - 1):
        @pl.when(_cid(t) < NCHUNK)
        def _():
            _start_idx(t, t)

    @pl.when(_cid(0) < NCHUNK)
    def _():
        _wait_idx(0, 0)
        _start_gather(0, 0)

    def _step(j, _):
        br = lax.rem(j, NBUF)
        nbr = lax.rem(j + 1, NBUF)
        bi = lax.rem(j, NIB)
        nbi = lax.rem(j + 1, NIB)
        pbi = lax.rem(j + NIB - 1, NIB)

        @pl.when(_cid(j + 1) < NCHUNK)
        def _():
            @pl.when(j >= 1)
            def _():
                _wait_scat(pbi, nbr)
            _wait_idx(j + 1, nbi)
            _start_gather(nbi, nbr)

        @pl.when(_cid(j) < NCHUNK)
        def _():
            _wait_gather(bi, br)
            _start_scat(bi, br)

        @pl.when(_cid(j + 2) < NCHUNK)
        def _():
            _start_idx(j + 2, pbi)

        return 0

    lax.fori_loop(0, NJ, _step, 0)

    def _drain(j, _):
        @pl.when(jnp.logical_and(_cid(j) < NCHUNK,
                                 _cid(j + 2) >= NCHUNK))
        def _():
            _wait_scat(lax.rem(j, NIB), lax.rem(j, NBUF))
        return 0

    lax.fori_loop(NJ - 3, NJ, _drain, 0)

    plsc.subcore_barrier()
    pltpu.sync_copy(acc.at[pl.ds(s * RPT, RPT)],
                    agg_out.at[c, pl.ds(s * RPT, RPT)])


_agg_call = pl.kernel(
    _agg_body,
    out_type=jax.ShapeDtypeStruct((NC, NPAD, D), jnp.float32),
    mesh=_mesh,
    scratch_types=[
        pltpu.VMEM((NIB, 2, CH), jnp.int32),
        pltpu.VMEM((NBUF, CH, D), jnp.float32),
        pltpu.VMEM_SHARED((NPAD, D), jnp.float32),
        pltpu.SemaphoreType.DMA((NIB,)),
        pltpu.SemaphoreType.DMA((NBUF,)),
        pltpu.SemaphoreType.DMA((NBUF,)),
    ],
    compiler_params=_sc_params,
)


_RB = 400


def _mm_body(f_ref, c0_ref, c1_ref, w_ref, o_ref):
    ns = 1.0 / jnp.sqrt(jnp.maximum(c0_ref[0] + c1_ref[0], 1.0))
    o_ref[...] = jnp.dot(f_ref[...] * ns, w_ref[...],
                         preferred_element_type=jnp.float32,
                         precision=lax.Precision.HIGHEST)


_mm_call = pl.pallas_call(
    _mm_body,
    grid=(N // _RB,),
    in_specs=[
        pl.BlockSpec((_RB, D), lambda i: (i, 0)),
        pl.BlockSpec((1, _RB, 1), lambda i: (0, i, 0)),
        pl.BlockSpec((1, _RB, 1), lambda i: (1, i, 0)),
        pl.BlockSpec((D, D), lambda i: (0, 0)),
    ],
    out_specs=pl.BlockSpec((_RB, D), lambda i: (i, 0)),
    out_shape=jax.ShapeDtypeStruct((N, D), jnp.float32),
)


def _fin_body(a_ref, b_ref, c0_ref, c1_ref, bias_ref, o_ref):
    nd = 1.0 / jnp.sqrt(jnp.maximum(c0_ref[0] + c1_ref[0], 1.0))
    o_ref[...] = (a_ref[0] + b_ref[0]) * nd + bias_ref[...]


_fin_call = pl.pallas_call(
    _fin_body,
    grid=(N // _RB,),
    in_specs=[
        pl.BlockSpec((1, _RB, D), lambda i: (0, i, 0)),
        pl.BlockSpec((1, _RB, D), lambda i: (1, i, 0)),
        pl.BlockSpec((1, _RB, 1), lambda i: (0, i, 0)),
        pl.BlockSpec((1, _RB, 1), lambda i: (1, i, 0)),
        pl.BlockSpec((1, D), lambda i: (0, 0)),
    ],
    out_specs=pl.BlockSpec((_RB, D), lambda i: (i, 0)),
    out_shape=jax.ShapeDtypeStruct((N, D), jnp.float32),
)


def kernel(feat, edge_index, weight, bias):
    ei3 = edge_index.reshape(2, NCHUNK, CH)
    scnt, dcnt = _deg_call(ei3)
    q = _mm_call(feat, scnt.reshape(NC, NPAD, 1), scnt.reshape(NC, NPAD, 1),
                 weight)
    agg = _agg_call(q, ei3)
    return _fin_call(agg, agg, dcnt.reshape(NC, NPAD, 1),
                     dcnt.reshape(NC, NPAD, 1), bias.reshape(1, D))


# speedup vs baseline: 14.1951x; 1.1992x over previous
import functools

import jax
import jax.numpy as jnp
from jax import lax
from jax.experimental import pallas as pl
from jax.experimental.pallas import tpu as pltpu
from jax.experimental.pallas import tpu_sc as plsc

N = 10000
E = 320000
D = 128

NC = 2
NS = 16
NW = NC * NS
EPW = E // NW
CH = 128
NFULL = EPW // CH
REM = EPW - NFULL * CH
NPAD = 10240
RPT = NPAD // NS
V = 16

_mesh = plsc.VectorSubcoreMesh(core_axis_name="c", subcore_axis_name="s")
_sc_params = pltpu.CompilerParams(needs_layout_passes=False)


def _zero_vmem(ref, n):
    zeros = jnp.zeros((V,), jnp.float32)

    def _z(i, _):
        ref[pl.ds(i * V, V)] = zeros
        return 0

    lax.fori_loop(0, n // V, _z, 0)


NCHUNK = E // CH
NJ = -(-NCHUNK // NW)
DEG_NBUF = 4


def _deg_body(ei_hbm, z_hbm, scnt_out, dcnt_out, idxb, ones, acc, acc_d, isem):
    c = lax.axis_index("c")
    s = lax.axis_index("s")
    w = s * NC + c

    one = jnp.ones((V,), jnp.float32)

    def _o(i, _):
        ones[pl.ds(i * V, V)] = one
        return 0

    lax.fori_loop(0, CH // V, _o, 0)
    pltpu.sync_copy(z_hbm, acc.at[pl.ds(s * RPT, RPT)])
    pltpu.sync_copy(z_hbm, acc_d.at[pl.ds(s * RPT, RPT)])
    plsc.subcore_barrier()

    def _cid(j):
        return w + j * NW

    def _start_idx(j, b):
        pltpu.async_copy(ei_hbm.at[:, pl.ds(_cid(j) * CH, CH)], idxb.at[b],
                         isem.at[b])

    def _wait_idx(j, b):
        pltpu.make_async_copy(ei_hbm.at[:, pl.ds(_cid(j) * CH, CH)],
                              idxb.at[b], isem.at[b]).wait()

    for t in range(DEG_NBUF):
        @pl.when(_cid(t) < NCHUNK)
        def _():
            _start_idx(t, t)

    def _step(j, _):
        b = lax.rem(j, DEG_NBUF)

        @pl.when(_cid(j) < NCHUNK)
        def _():
            _wait_idx(j, b)
            pltpu.sync_copy(ones, acc.at[idxb.at[b, 0]], add=True)
            pltpu.sync_copy(ones, acc_d.at[idxb.at[b, 1]], add=True)

        @pl.when(_cid(j + DEG_NBUF) < NCHUNK)
        def _():
            _start_idx(j + DEG_NBUF, b)

        return 0

    lax.fori_loop(0, NJ, _step, 0)

    plsc.subcore_barrier()
    pltpu.sync_copy(acc.at[pl.ds(s * RPT, RPT)],
                    scnt_out.at[c, pl.ds(s * RPT, RPT)])
    pltpu.sync_copy(acc_d.at[pl.ds(s * RPT, RPT)],
                    dcnt_out.at[c, pl.ds(s * RPT, RPT)])


_deg_call = pl.kernel(
    _deg_body,
    out_type=(
        jax.ShapeDtypeStruct((NC, NPAD), jnp.float32),
        jax.ShapeDtypeStruct((NC, NPAD), jnp.float32),
    ),
    mesh=_mesh,
    scratch_types=[
        pltpu.VMEM((DEG_NBUF, 2, CH), jnp.int32),
        pltpu.VMEM((CH,), jnp.float32),
        pltpu.VMEM_SHARED((NPAD,), jnp.float32),
        pltpu.VMEM_SHARED((NPAD,), jnp.float32),
        pltpu.SemaphoreType.DMA((DEG_NBUF,)),
    ],
    compiler_params=_sc_params,
)


NBUF = 2
NIB = 3


def _agg_body(q_hbm, ei_hbm, agg_out,
              idxb, rows, acc, isem, gsem, ssem):
    c = lax.axis_index("c")
    s = lax.axis_index("s")
    w = s * NC + c

    zeros = jnp.zeros((V,), jnp.float32)

    def _z0(i, _):
        def _z1(j, _):
            rows[0, i, pl.ds(j * V, V)] = zeros
            return 0
        lax.fori_loop(0, D // V, _z1, 0)
        return 0

    lax.fori_loop(0, CH, _z0, 0)
    for k in range(RPT // CH):
        pltpu.sync_copy(rows.at[0], acc.at[pl.ds(s * RPT + k * CH, CH)])
    plsc.subcore_barrier()

    def _cid(j):
        return w + j * NW

    def _start_idx(j, bi):
        pltpu.async_copy(ei_hbm.at[:, pl.ds(_cid(j) * CH, CH)], idxb.at[bi],
                         isem.at[bi])

    def _wait_idx(j, bi):
        pltpu.make_async_copy(ei_hbm.at[:, pl.ds(_cid(j) * CH, CH)],
                              idxb.at[bi], isem.at[bi]).wait()

    def _start_gather(bi, br):
        pltpu.async_copy(q_hbm.at[idxb.at[bi, 0]], rows.at[br], gsem.at[br])

    def _wait_gather(bi, br):
        pltpu.make_async_copy(q_hbm.at[idxb.at[bi, 0]], rows.at[br],
                              gsem.at[br]).wait()

    def _start_scat(bi, br):
        pltpu.async_copy(rows.at[br], acc.at[idxb.at[bi, 1]], ssem.at[br],
                         add=True)

    def _wait_scat(bi, br):
        pltpu.make_async_copy(rows.at[br], acc.at[idxb.at[bi, 1]],
                              ssem.at[br]).wait()

    for t in range(NIB - 1):
        @pl.when(_cid(t) < NCHUNK)
        def _():
            _start_idx(t, t)

    @pl.when(_cid(0) < NCHUNK)
    def _():
        _wait_idx(0, 0)
        _start_gather(0, 0)

    def _step(j, _):
        br = lax.rem(j, NBUF)
        nbr = lax.rem(j + 1, NBUF)
        bi = lax.rem(j, NIB)
        nbi = lax.rem(j + 1, NIB)
        pbi = lax.rem(j + NIB - 1, NIB)

        @pl.when(_cid(j + 1) < NCHUNK)
        def _():
            @pl.when(j >= 1)
            def _():
                _wait_scat(pbi, nbr)
            _wait_idx(j + 1, nbi)
            _start_gather(nbi, nbr)

        @pl.when(_cid(j) < NCHUNK)
        def _():
            _wait_gather(bi, br)
            _start_scat(bi, br)

        @pl.when(_cid(j + 2) < NCHUNK)
        def _():
            _start_idx(j + 2, pbi)

        return 0

    lax.fori_loop(0, NJ, _step, 0)

    def _drain(j, _):
        @pl.when(jnp.logical_and(_cid(j) < NCHUNK,
                                 _cid(j + 2) >= NCHUNK))
        def _():
            _wait_scat(lax.rem(j, NIB), lax.rem(j, NBUF))
        return 0

    lax.fori_loop(NJ - 3, NJ, _drain, 0)

    plsc.subcore_barrier()
    pltpu.sync_copy(acc.at[pl.ds(s * RPT, RPT)],
                    agg_out.at[c, pl.ds(s * RPT, RPT)])


_agg_call = pl.kernel(
    _agg_body,
    out_type=jax.ShapeDtypeStruct((NC, NPAD, D), jnp.float32),
    mesh=_mesh,
    scratch_types=[
        pltpu.VMEM((NIB, 2, CH), jnp.int32),
        pltpu.VMEM((NBUF, CH, D), jnp.float32),
        pltpu.VMEM_SHARED((NPAD, D), jnp.float32),
        pltpu.SemaphoreType.DMA((NIB,)),
        pltpu.SemaphoreType.DMA((NBUF,)),
        pltpu.SemaphoreType.DMA((NBUF,)),
    ],
    compiler_params=_sc_params,
)


_RB = 2048


def _mm_body(f_ref, c_ref, w_ref, o_ref):
    cnt = jnp.transpose(jnp.sum(c_ref[...], axis=0, keepdims=True))
    ns = 1.0 / jnp.sqrt(jnp.maximum(cnt, 1.0))
    o_ref[...] = jnp.dot(f_ref[...] * ns, w_ref[...],
                         preferred_element_type=jnp.float32)


_mm_call = pl.pallas_call(
    _mm_body,
    grid=(NPAD // _RB,),
    in_specs=[
        pl.BlockSpec((_RB, D), lambda i: (i, 0)),
        pl.BlockSpec((NC, _RB), lambda i: (0, i)),
        pl.BlockSpec((D, D), lambda i: (0, 0)),
    ],
    out_specs=pl.BlockSpec((_RB, D), lambda i: (i, 0)),
    out_shape=jax.ShapeDtypeStruct((N, D), jnp.float32),
)


def _fin_body(a_ref, b_ref, c_ref, bias_ref, o_ref):
    cnt = jnp.transpose(jnp.sum(c_ref[...], axis=0, keepdims=True))
    nd = 1.0 / jnp.sqrt(jnp.maximum(cnt, 1.0))
    o_ref[...] = (a_ref[0] + b_ref[0]) * nd + bias_ref[...]


_fin_call = pl.pallas_call(
    _fin_body,
    grid=(NPAD // _RB,),
    in_specs=[
        pl.BlockSpec((1, _RB, D), lambda i: (0, i, 0)),
        pl.BlockSpec((1, _RB, D), lambda i: (1, i, 0)),
        pl.BlockSpec((NC, _RB), lambda i: (0, i)),
        pl.BlockSpec((1, D), lambda i: (0, 0)),
    ],
    out_specs=pl.BlockSpec((_RB, D), lambda i: (i, 0)),
    out_shape=jax.ShapeDtypeStruct((N, D), jnp.float32),
)


def kernel(feat, edge_index, weight, bias):
    zcol = jnp.zeros((RPT,), jnp.float32)
    scnt, dcnt = _deg_call(edge_index, zcol)
    q = _mm_call(feat, scnt, weight)
    agg = _agg_call(q, edge_index)
    return _fin_call(agg, agg, dcnt, bias.reshape(1, D))


# speedup vs baseline: 14.4111x; 1.0152x over previous
import functools

import jax
import jax.numpy as jnp
from jax import lax
from jax.experimental import pallas as pl
from jax.experimental.pallas import tpu as pltpu
from jax.experimental.pallas import tpu_sc as plsc

N = 10000
E = 320000
D = 128

NC = 2
NS = 16
NW = NC * NS
EPW = E // NW
CH = 128
NFULL = EPW // CH
REM = EPW - NFULL * CH
NPAD = 10240
RPT = NPAD // NS
V = 16

_mesh = plsc.VectorSubcoreMesh(core_axis_name="c", subcore_axis_name="s")
_sc_params = pltpu.CompilerParams(needs_layout_passes=False)


def _zero_vmem(ref, n):
    zeros = jnp.zeros((V,), jnp.float32)

    def _z(i, _):
        ref[pl.ds(i * V, V)] = zeros
        return 0

    lax.fori_loop(0, n // V, _z, 0)


NCHUNK = E // CH
NJ = -(-NCHUNK // NW)
DEG_NBUF = 4


def _deg_body(ei_hbm, z_hbm, scnt_out, dcnt_out, idxb, ones, acc, acc_d, isem, ssem, dsem):
    c = lax.axis_index("c")
    s = lax.axis_index("s")
    w = s * NC + c

    one = jnp.ones((V,), jnp.float32)

    def _o(i, _):
        ones[pl.ds(i * V, V)] = one
        return 0

    lax.fori_loop(0, CH // V, _o, 0)
    pltpu.sync_copy(z_hbm, acc.at[pl.ds(s * RPT, RPT)])
    pltpu.sync_copy(z_hbm, acc_d.at[pl.ds(s * RPT, RPT)])
    plsc.subcore_barrier()

    def _cid(j):
        return w + j * NW

    def _start_idx(j, b):
        pltpu.async_copy(ei_hbm.at[:, pl.ds(_cid(j) * CH, CH)], idxb.at[b],
                         isem.at[b])

    def _wait_idx(j, b):
        pltpu.make_async_copy(ei_hbm.at[:, pl.ds(_cid(j) * CH, CH)],
                              idxb.at[b], isem.at[b]).wait()

    def _start_scat(b):
        pltpu.async_copy(ones, acc.at[idxb.at[b, 0]], ssem.at[b], add=True)
        pltpu.async_copy(ones, acc_d.at[idxb.at[b, 1]], dsem.at[b], add=True)

    def _wait_scat(b):
        pltpu.make_async_copy(ones, acc.at[idxb.at[b, 0]],
                              ssem.at[b]).wait()
        pltpu.make_async_copy(ones, acc_d.at[idxb.at[b, 1]],
                              dsem.at[b]).wait()

    for t in range(2):
        @pl.when(_cid(t) < NCHUNK)
        def _():
            _start_idx(t, t)

    def _step(j, _):
        b = lax.rem(j, DEG_NBUF)
        fb = lax.rem(j + 2, DEG_NBUF)

        @pl.when(_cid(j + 2) < NCHUNK)
        def _():
            @pl.when(j >= 2)
            def _():
                _wait_scat(fb)
            _start_idx(j + 2, fb)

        @pl.when(_cid(j) < NCHUNK)
        def _():
            _wait_idx(j, b)
            _start_scat(b)

        return 0

    lax.fori_loop(0, NJ, _step, 0)

    def _deg_drain(j, _):
        @pl.when(jnp.logical_and(_cid(j) < NCHUNK,
                                 _cid(j + 4) >= NCHUNK))
        def _():
            _wait_scat(lax.rem(j, DEG_NBUF))
        return 0

    lax.fori_loop(NJ - 5, NJ, _deg_drain, 0)

    plsc.subcore_barrier()
    pltpu.sync_copy(acc.at[pl.ds(s * RPT, RPT)],
                    scnt_out.at[c, pl.ds(s * RPT, RPT)])
    pltpu.sync_copy(acc_d.at[pl.ds(s * RPT, RPT)],
                    dcnt_out.at[c, pl.ds(s * RPT, RPT)])


_deg_call = pl.kernel(
    _deg_body,
    out_type=(
        jax.ShapeDtypeStruct((NC, NPAD), jnp.float32),
        jax.ShapeDtypeStruct((NC, NPAD), jnp.float32),
    ),
    mesh=_mesh,
    scratch_types=[
        pltpu.VMEM((DEG_NBUF, 2, CH), jnp.int32),
        pltpu.VMEM((CH,), jnp.float32),
        pltpu.VMEM_SHARED((NPAD,), jnp.float32),
        pltpu.VMEM_SHARED((NPAD,), jnp.float32),
        pltpu.SemaphoreType.DMA((DEG_NBUF,)),
        pltpu.SemaphoreType.DMA((DEG_NBUF,)),
        pltpu.SemaphoreType.DMA((DEG_NBUF,)),
    ],
    compiler_params=_sc_params,
)


NBUF = 2
NIB = 3


def _agg_body(q_hbm, ei_hbm, agg_out,
              idxb, rows, acc, isem, gsem, ssem):
    c = lax.axis_index("c")
    s = lax.axis_index("s")
    w = s * NC + c

    zeros = jnp.zeros((V,), jnp.float32)

    def _z0(i, _):
        def _z1(j, _):
            rows[0, i, pl.ds(j * V, V)] = zeros
            return 0
        lax.fori_loop(0, D // V, _z1, 0)
        return 0

    lax.fori_loop(0, CH, _z0, 0)
    for k in range(RPT // CH):
        pltpu.sync_copy(rows.at[0], acc.at[pl.ds(s * RPT + k * CH, CH)])
    plsc.subcore_barrier()

    def _cid(j):
        return w + j * NW

    def _start_idx(j, bi):
        pltpu.async_copy(ei_hbm.at[:, pl.ds(_cid(j) * CH, CH)], idxb.at[bi],
                         isem.at[bi])

    def _wait_idx(j, bi):
        pltpu.make_async_copy(ei_hbm.at[:, pl.ds(_cid(j) * CH, CH)],
                              idxb.at[bi], isem.at[bi]).wait()

    def _start_gather(bi, br):
        pltpu.async_copy(q_hbm.at[idxb.at[bi, 0]], rows.at[br], gsem.at[br])

    def _wait_gather(bi, br):
        pltpu.make_async_copy(q_hbm.at[idxb.at[bi, 0]], rows.at[br],
                              gsem.at[br]).wait()

    def _start_scat(bi, br):
        pltpu.async_copy(rows.at[br], acc.at[idxb.at[bi, 1]], ssem.at[br],
                         add=True)

    def _wait_scat(bi, br):
        pltpu.make_async_copy(rows.at[br], acc.at[idxb.at[bi, 1]],
                              ssem.at[br]).wait()

    for t in range(NIB - 1):
        @pl.when(_cid(t) < NCHUNK)
        def _():
            _start_idx(t, t)

    @pl.when(_cid(0) < NCHUNK)
    def _():
        _wait_idx(0, 0)
        _start_gather(0, 0)

    def _step(j, _):
        br = lax.rem(j, NBUF)
        nbr = lax.rem(j + 1, NBUF)
        bi = lax.rem(j, NIB)
        nbi = lax.rem(j + 1, NIB)
        pbi = lax.rem(j + NIB - 1, NIB)

        @pl.when(_cid(j + 1) < NCHUNK)
        def _():
            @pl.when(j >= 1)
            def _():
                _wait_scat(pbi, nbr)
            _wait_idx(j + 1, nbi)
            _start_gather(nbi, nbr)

        @pl.when(_cid(j) < NCHUNK)
        def _():
            _wait_gather(bi, br)
            _start_scat(bi, br)

        @pl.when(_cid(j + 2) < NCHUNK)
        def _():
            _start_idx(j + 2, pbi)

        return 0

    lax.fori_loop(0, NJ, _step, 0)

    def _drain(j, _):
        @pl.when(jnp.logical_and(_cid(j) < NCHUNK,
                                 _cid(j + 2) >= NCHUNK))
        def _():
            _wait_scat(lax.rem(j, NIB), lax.rem(j, NBUF))
        return 0

    lax.fori_loop(NJ - 3, NJ, _drain, 0)

    plsc.subcore_barrier()
    pltpu.sync_copy(acc.at[pl.ds(s * RPT, RPT)],
                    agg_out.at[c, pl.ds(s * RPT, RPT)])


_agg_call = pl.kernel(
    _agg_body,
    out_type=jax.ShapeDtypeStruct((NC, NPAD, D), jnp.float32),
    mesh=_mesh,
    scratch_types=[
        pltpu.VMEM((NIB, 2, CH), jnp.int32),
        pltpu.VMEM((NBUF, CH, D), jnp.float32),
        pltpu.VMEM_SHARED((NPAD, D), jnp.float32),
        pltpu.SemaphoreType.DMA((NIB,)),
        pltpu.SemaphoreType.DMA((NBUF,)),
        pltpu.SemaphoreType.DMA((NBUF,)),
    ],
    compiler_params=_sc_params,
)


_RB = 2048


def _mm_body(f_ref, c_ref, w_ref, o_ref):
    cnt = jnp.transpose(jnp.sum(c_ref[...], axis=0, keepdims=True))
    ns = 1.0 / jnp.sqrt(jnp.maximum(cnt, 1.0))
    o_ref[...] = jnp.dot(f_ref[...] * ns, w_ref[...],
                         preferred_element_type=jnp.float32)


_mm_call = pl.pallas_call(
    _mm_body,
    grid=(NPAD // _RB,),
    in_specs=[
        pl.BlockSpec((_RB, D), lambda i: (i, 0)),
        pl.BlockSpec((NC, _RB), lambda i: (0, i)),
        pl.BlockSpec((D, D), lambda i: (0, 0)),
    ],
    out_specs=pl.BlockSpec((_RB, D), lambda i: (i, 0)),
    out_shape=jax.ShapeDtypeStruct((N, D), jnp.float32),
)


def _fin_body(a_ref, b_ref, c_ref, bias_ref, o_ref):
    cnt = jnp.transpose(jnp.sum(c_ref[...], axis=0, keepdims=True))
    nd = 1.0 / jnp.sqrt(jnp.maximum(cnt, 1.0))
    o_ref[...] = (a_ref[0] + b_ref[0]) * nd + bias_ref[...]


_fin_call = pl.pallas_call(
    _fin_body,
    grid=(NPAD // _RB,),
    in_specs=[
        pl.BlockSpec((1, _RB, D), lambda i: (0, i, 0)),
        pl.BlockSpec((1, _RB, D), lambda i: (1, i, 0)),
        pl.BlockSpec((NC, _RB), lambda i: (0, i)),
        pl.BlockSpec((1, D), lambda i: (0, 0)),
    ],
    out_specs=pl.BlockSpec((_RB, D), lambda i: (i, 0)),
    out_shape=jax.ShapeDtypeStruct((N, D), jnp.float32),
)


def kernel(feat, edge_index, weight, bias):
    zcol = jnp.zeros((RPT,), jnp.float32)
    scnt, dcnt = _deg_call(edge_index, zcol)
    q = _mm_call(feat, scnt, weight)
    agg = _agg_call(q, edge_index)
    return _fin_call(agg, agg, dcnt, bias.reshape(1, D))
